# baseline (speedup 1.0000x reference)
import jax
import jax.numpy as jnp
from jax import lax
from jax.experimental import pallas as pl
from jax.experimental.pallas import tpu as pltpu
from jax.experimental.pallas import tpu_sc as plsc

N = 100000
D = 128
D4 = 4 * D
NUM_GRAPHS = 128
NACC = 3 * NUM_GRAPHS
NCOMP = 3
NC = 2
NS = 16
NW = NC * NS
L = 16
NQ = N // 4
TQ = 40
QSUB = NQ // TQ
SUB_LO = QSUB // NW
SUB_HI = SUB_LO + 1
N_HI = QSUB - NW * SUB_LO
T = 125
NSUB = N // T
CNT_PER_S = NSUB // NS
GRP = D // L
NBUF = 2
IDS_Q_ROWS = QSUB + 8


def _sc_body(f_hbm, idsq0_hbm, idsq1_hbm, idsq2_hbm, idsq3_hbm,
             ids2d_hbm, out_hbm,
             acc, corr, cnt,
             ids3, idq1, idq2, idq3, ids_cn, buf_a, buf_b, corr_l,
             zbuf, czbuf, cloc, identr, idx3, fbuf, fcorr, cbuf,
             sem_g, sem_s):
    c = lax.axis_index("c")
    s = lax.axis_index("s")
    wid = s * NC + c
    base_sub = wid * SUB_LO + jnp.minimum(wid, N_HI)
    nsub = SUB_LO + jnp.where(wid < N_HI, 1, 0)
    nunit = NCOMP * nsub

    zero = jnp.zeros((L,), jnp.float32)
    for r in range(8):
        for g in range(D4 // L):
            zbuf[r, pl.ds(g * L, L)] = zero
        for g in range(GRP):
            fbuf[r, pl.ds(g * L, L)] = zero
        czbuf[r, :] = zero
    for q in range(3):
        pltpu.sync_copy(zbuf, acc.at[pl.ds(s * 24 + q * 8, 8)])
        pltpu.sync_copy(fbuf, corr.at[pl.ds(s * 24 + q * 8, 8)])
    pltpu.sync_copy(czbuf, cnt.at[pl.ds(s * 8, 8)])

    def zero_corr(r, carry):
        for g in range(GRP):
            corr_l[r, pl.ds(g * L, L)] = zero
        return carry

    lax.fori_loop(0, NACC, zero_corr, 0)

    lanes = lax.iota(jnp.int32, L)
    ones = jnp.full((L,), 1.0, jnp.float32)
    for r in range(NUM_GRAPHS):
        cloc[r, :] = zero
    for g in range(NUM_GRAPHS // L):
        identr[0, pl.ds(g * L, L)] = lanes + jnp.full((L,), g * L, jnp.int32)
        for q in range(NCOMP):
            idx3[q, pl.ds(g * L, L)] = (
                lanes + jnp.full((L,), q * NUM_GRAPHS + g * L, jnp.int32))
    tail_msk = lanes >= jnp.full((L,), 3, jnp.int32)
    for h in range(2):
        pltpu.sync_copy(
            ids2d_hbm.at[pl.ds(s * CNT_PER_S + h * (CNT_PER_S // 2),
                               CNT_PER_S // 2)], ids_cn)
        for j in range(CNT_PER_S // 2):
            for g in range(T // L):
                idv = ids_cn[j, pl.ds(g * L, L)]
                plsc.addupdate_scatter(cloc, [idv, lanes], ones)
            idv = ids_cn[j, pl.ds(T - L, L)]
            plsc.addupdate_scatter(cloc, [idv, lanes], ones, mask=tail_msk)

    pltpu.sync_copy(idsq0_hbm.at[pl.ds(base_sub, SUB_HI)], ids3.at[0])
    pltpu.sync_copy(idsq1_hbm.at[pl.ds(base_sub, SUB_HI)], idq1)
    pltpu.sync_copy(idsq2_hbm.at[pl.ds(base_sub, SUB_HI)], idq2)
    pltpu.sync_copy(idsq3_hbm.at[pl.ds(base_sub, SUB_HI)], idq3)
    for j in range(SUB_HI):
        for st in (0, 16, 24):
            base_v = ids3[0, j, pl.ds(st, L)]
            ids3[1, j, pl.ds(st, L)] = base_v + jnp.full(
                (L,), NUM_GRAPHS, jnp.int32)
            ids3[2, j, pl.ds(st, L)] = base_v + jnp.full(
                (L,), 2 * NUM_GRAPHS, jnp.int32)

    bufs = (buf_a, buf_b)
    idqs = (idq1, idq2, idq3)

    def gslice(u):
        k = lax.div(u, nsub)
        j = lax.rem(u, nsub)
        return k, j

    def start_gather_b(u, b):
        k, j = gslice(u)
        pltpu.async_copy(
            f_hbm.at[pl.ds(k * NQ + (base_sub + j) * TQ, TQ), :],
            bufs[b], sem_g)

    start_gather_b(jnp.int32(0), 0)
    start_gather_b(jnp.int32(1), 1)

    def step(t, carry):
        for b in range(NBUF):
            u = t * NBUF + b

            @pl.when(u < nunit)
            def _():
                k, j = gslice(u)
                pltpu.make_async_copy(
                    f_hbm.at[pl.ds(0, TQ), :], bufs[b], sem_g).wait()
                kb = jnp.full((L,), k * NUM_GRAPHS, jnp.int32)
                for blk, st in ((0, 0), (1, 16), (2, 24)):
                    ev0 = ids3[0, j, pl.ds(st, L)]
                    anyv = jnp.full((L,), 0, jnp.int32)
                    for r in range(3):
                        anyv = anyv + (ev0 != idqs[r][j, pl.ds(st, L)]
                                       ).astype(jnp.int32)
                    if blk == 2:
                        anyv = anyv * (lanes >= jnp.full(
                            (L,), 8, jnp.int32)).astype(jnp.int32)

                    @pl.when(jnp.any(anyv > jnp.full((L,), 0, jnp.int32)))
                    def _(st=st, b=b, blk=blk, j=j, kb=kb):
                        l0 = 8 if blk == 2 else 0

                        def corr_one(l, carry):
                            lsel = jnp.full((L,), l, jnp.int32)
                            ev = ids3[0, j, pl.ds(st, L)] + kb
                            id_ev = ev.at[lsel].get(
                                mode="promise_in_bounds")
                            p = st + l
                            for r in range(3):
                                od = idqs[r][j, pl.ds(st, L)] + kb
                                m = (ev != od).astype(jnp.int32).at[
                                    lsel].get(mode="promise_in_bounds"
                                              ) > jnp.full((L,), 0,
                                                           jnp.int32)
                                id_od = od.at[lsel].get(
                                    mode="promise_in_bounds")
                                for g in range(GRP):
                                    v = bufs[b][
                                        p, pl.ds((r + 1) * D + g * L, L)]
                                    colg = lanes + jnp.full(
                                        (L,), g * L, jnp.int32)
                                    plsc.addupdate_scatter(
                                        corr_l, [id_od, colg], v, mask=m)
                                    plsc.addupdate_scatter(
                                        corr_l, [id_ev, colg], -v, mask=m)
                            return carry

                        lax.fori_loop(l0, L, corr_one, 0)
                pltpu.async_copy(
                    bufs[b], acc.at[ids3.at[k, j]], sem_s,
                    add=True).wait()

                @pl.when(u + NBUF < nunit)
                def _():
                    start_gather_b(u + NBUF, b)
        return carry

    nsteps = lax.div(nunit + NBUF - 1, NBUF)
    lax.fori_loop(0, nsteps, step, 0)

    for q in range(NCOMP):
        pltpu.sync_copy(corr_l.at[pl.ds(q * NUM_GRAPHS, NUM_GRAPHS)],
                        corr.at[idx3.at[q]], add=True)
    pltpu.sync_copy(cloc, cnt.at[identr.at[0]], add=True)

    plsc.subcore_barrier()

    pltpu.sync_copy(cnt.at[pl.ds(s * 8, 8)], cbuf)
    for k in range(NCOMP):
        pltpu.sync_copy(acc.at[pl.ds(k * NUM_GRAPHS + s * 8, 8)], zbuf)
        pltpu.sync_copy(corr.at[pl.ds(k * NUM_GRAPHS + s * 8, 8)], fcorr)
        for r in range(8):
            total = jnp.sum(cbuf[r, :])
            rec = jnp.full((L,), 1.0, jnp.float32) / jnp.maximum(
                jnp.full((L,), total, jnp.float32),
                jnp.full((L,), 1.0, jnp.float32))
            for g in range(GRP):
                acc4 = (zbuf[r, pl.ds(g * L, L)]
                        + zbuf[r, pl.ds(D + g * L, L)]
                        + zbuf[r, pl.ds(2 * D + g * L, L)]
                        + zbuf[r, pl.ds(3 * D + g * L, L)]
                        + fcorr[r, pl.ds(g * L, L)])
                fbuf[r, pl.ds(g * L, L)] = acc4 * rec
        pltpu.sync_copy(fbuf, out_hbm.at[c, k, pl.ds(s * 8, 8), :])


@jax.jit
def _gavg_pool(f4, iq0, iq1, iq2, iq3, ids2d):
    mesh = plsc.VectorSubcoreMesh(core_axis_name="c", subcore_axis_name="s")
    return pl.kernel(
        _sc_body,
        out_type=jax.ShapeDtypeStruct((NC, NCOMP, NUM_GRAPHS, D),
                                      jnp.float32),
        mesh=mesh,
        compiler_params=pltpu.CompilerParams(
            use_tc_tiling_on_sc=False, needs_layout_passes=False),
        scratch_types=[
            pltpu.VMEM_SHARED((NACC, D4), jnp.float32),
            pltpu.VMEM_SHARED((NACC, D), jnp.float32),
            pltpu.VMEM_SHARED((NUM_GRAPHS, L), jnp.float32),
            pltpu.VMEM((NCOMP, SUB_HI, TQ), jnp.int32),
            pltpu.VMEM((SUB_HI, TQ), jnp.int32),
            pltpu.VMEM((SUB_HI, TQ), jnp.int32),
            pltpu.VMEM((SUB_HI, TQ), jnp.int32),
            pltpu.VMEM((CNT_PER_S // 2, T), jnp.int32),
            pltpu.VMEM((TQ, D4), jnp.float32),
            pltpu.VMEM((TQ, D4), jnp.float32),
            pltpu.VMEM((NACC, D), jnp.float32),
            pltpu.VMEM((8, D4), jnp.float32),
            pltpu.VMEM((8, L), jnp.float32),
            pltpu.VMEM((NUM_GRAPHS, L), jnp.float32),
            pltpu.VMEM((1, NUM_GRAPHS), jnp.int32),
            pltpu.VMEM((NCOMP, NUM_GRAPHS), jnp.int32),
            pltpu.VMEM((8, D), jnp.float32),
            pltpu.VMEM((8, D), jnp.float32),
            pltpu.VMEM((8, L), jnp.float32),
            pltpu.SemaphoreType.DMA,
            pltpu.SemaphoreType.DMA,
        ],
    )(f4, iq0, iq1, iq2, iq3, ids2d)


def kernel(features_1, segment_ids):
    f4 = jnp.transpose(features_1, (2, 0, 1)).reshape(NCOMP * NQ, D4)
    ids = segment_ids.astype(jnp.int32)
    pad = IDS_Q_ROWS * TQ - NQ
    iqs = [jnp.pad(ids[r::4], (0, pad),
                   constant_values=NUM_GRAPHS - 1).reshape(IDS_Q_ROWS, TQ)
           for r in range(4)]
    raw = _gavg_pool(f4, iqs[0], iqs[1], iqs[2], iqs[3],
                     ids.reshape(NSUB, T))
    comb = raw[0] + raw[1]
    return jnp.transpose(comb, (1, 0, 2)).reshape(NUM_GRAPHS, NCOMP * D)


# speedup vs baseline: 1.1743x; 1.1743x over previous
import jax
import jax.numpy as jnp
from jax import lax
from jax.experimental import pallas as pl
from jax.experimental.pallas import tpu as pltpu
from jax.experimental.pallas import tpu_sc as plsc

N = 100000
D = 128
D2 = 2 * D
NUM_GRAPHS = 128
NCOMP = 3
NC = 2
NS = 16
NW = NC * NS
L = 16
NP = N // 2
TP = 80
PSUB = NP // TP
SUB_LO = PSUB // NW
SUB_HI = SUB_LO + 1
N_HI = PSUB - NW * SUB_LO
T = 125
NSUB = N // T
CNT_PER_S = NSUB // NS
GRP = D // L
GRP2 = D2 // L
NBUF = 2
IDS_P_ROWS = PSUB + 8


def _sc_body(f_hbm, idsp_hbm, idso_hbm, ids2d_hbm, out_hbm,
             acc, corr, cnt,
             ids3, ids_o, ids_cn, buf_a, buf_b, corr_l,
             zbuf, czbuf, cloc, identr, idx3, fbuf, fcorr, cbuf,
             sem_g, sem_s):
    c = lax.axis_index("c")
    s = lax.axis_index("s")
    wid = s * NC + c
    base_sub = wid * SUB_LO + jnp.minimum(wid, N_HI)
    nsub = SUB_LO + jnp.where(wid < N_HI, 1, 0)
    nunit = NCOMP * nsub

    zero = jnp.zeros((L,), jnp.float32)
    for r in range(8):
        for g in range(GRP2):
            zbuf[r, pl.ds(g * L, L)] = zero
        for g in range(GRP):
            fbuf[r, pl.ds(g * L, L)] = zero
        czbuf[r, :] = zero
    for q in range(3):
        pltpu.sync_copy(zbuf, acc.at[pl.ds(s * 24 + q * 8, 8)])
        pltpu.sync_copy(fbuf, corr.at[pl.ds(s * 24 + q * 8, 8)])
    pltpu.sync_copy(czbuf, cnt.at[pl.ds(s * 8, 8)])

    def zero_corr(r, carry):
        for g in range(GRP):
            corr_l[r, pl.ds(g * L, L)] = zero
        return carry

    lax.fori_loop(0, NCOMP * NUM_GRAPHS, zero_corr, 0)

    lanes = lax.iota(jnp.int32, L)
    ones = jnp.full((L,), 1.0, jnp.float32)
    for r in range(NUM_GRAPHS):
        cloc[r, :] = zero
    for g in range(NUM_GRAPHS // L):
        identr[0, pl.ds(g * L, L)] = lanes + jnp.full((L,), g * L, jnp.int32)
        for q in range(NCOMP):
            idx3[q, pl.ds(g * L, L)] = (
                lanes + jnp.full((L,), q * NUM_GRAPHS + g * L, jnp.int32))
    pltpu.sync_copy(ids2d_hbm.at[pl.ds(s * CNT_PER_S, CNT_PER_S)], ids_cn)
    tail_msk = lanes >= jnp.full((L,), 3, jnp.int32)
    for j in range(CNT_PER_S):
        for g in range(T // L):
            idv = ids_cn[j, pl.ds(g * L, L)]
            plsc.addupdate_scatter(cloc, [idv, lanes], ones)
        idv = ids_cn[j, pl.ds(T - L, L)]
        plsc.addupdate_scatter(cloc, [idv, lanes], ones, mask=tail_msk)

    pltpu.sync_copy(idsp_hbm.at[pl.ds(base_sub, SUB_HI)], ids3.at[0])
    pltpu.sync_copy(idso_hbm.at[pl.ds(base_sub, SUB_HI)], ids_o)
    for j in range(SUB_HI):
        for g in range(TP // L):
            base_v = ids3[0, j, pl.ds(g * L, L)]
            ids3[1, j, pl.ds(g * L, L)] = base_v + jnp.full(
                (L,), NUM_GRAPHS, jnp.int32)
            ids3[2, j, pl.ds(g * L, L)] = base_v + jnp.full(
                (L,), 2 * NUM_GRAPHS, jnp.int32)

    bufs = (buf_a, buf_b)

    def gslice(u):
        k = lax.div(u, nsub)
        j = lax.rem(u, nsub)
        return k, j

    def start_gather_b(u, b):
        k, j = gslice(u)
        pltpu.async_copy(
            f_hbm.at[pl.ds(k * NP + (base_sub + j) * TP, TP), :],
            bufs[b], sem_g)

    start_gather_b(jnp.int32(0), 0)
    start_gather_b(jnp.int32(1), 1)

    def step(t, carry):
        for b in range(NBUF):
            u = t * NBUF + b

            @pl.when(u < nunit)
            def _():
                k, j = gslice(u)
                pltpu.make_async_copy(
                    f_hbm.at[pl.ds(0, TP), :], bufs[b], sem_g).wait()
                sd = pltpu.async_copy(
                    bufs[b], acc.at[ids3.at[k, j]], sem_s, add=True)
                kb = jnp.full((L,), k * NUM_GRAPHS, jnp.int32)
                for blk in range(TP // L):
                    ev = ids3[0, j, pl.ds(blk * L, L)] + kb
                    od = ids_o[j, pl.ds(blk * L, L)] + kb
                    neq = (ev != od).astype(jnp.int32)

                    @pl.when(jnp.any(ev != od))
                    def _(blk=blk, b=b, ev=ev, od=od, neq=neq):
                        def corr_one(l, carry):
                            lsel = jnp.full((L,), l, jnp.int32)
                            m = neq.at[lsel].get(
                                mode="promise_in_bounds") > jnp.full(
                                    (L,), 0, jnp.int32)
                            id_ev = ev.at[lsel].get(
                                mode="promise_in_bounds")
                            id_od = od.at[lsel].get(
                                mode="promise_in_bounds")
                            p = blk * L + l
                            for g in range(GRP):
                                v = bufs[b][p, pl.ds(D + g * L, L)]
                                colg = lanes + jnp.full(
                                    (L,), g * L, jnp.int32)
                                plsc.addupdate_scatter(
                                    corr_l, [id_od, colg], v, mask=m)
                                plsc.addupdate_scatter(
                                    corr_l, [id_ev, colg], -v, mask=m)
                            return carry

                        lax.fori_loop(0, L, corr_one, 0)
                sd.wait()

                @pl.when(u + NBUF < nunit)
                def _():
                    start_gather_b(u + NBUF, b)
        return carry

    nsteps = lax.div(nunit + NBUF - 1, NBUF)
    lax.fori_loop(0, nsteps, step, 0)

    for q in range(NCOMP):
        pltpu.sync_copy(corr_l.at[pl.ds(q * NUM_GRAPHS, NUM_GRAPHS)],
                        corr.at[idx3.at[q]], add=True)
    pltpu.sync_copy(cloc, cnt.at[identr.at[0]], add=True)

    plsc.subcore_barrier()

    pltpu.sync_copy(cnt.at[pl.ds(s * 8, 8)], cbuf)
    for k in range(NCOMP):
        pltpu.sync_copy(acc.at[pl.ds(k * NUM_GRAPHS + s * 8, 8)], zbuf)
        pltpu.sync_copy(corr.at[pl.ds(k * NUM_GRAPHS + s * 8, 8)], fcorr)
        for r in range(8):
            total = jnp.sum(cbuf[r, :])
            rec = jnp.full((L,), 1.0, jnp.float32) / jnp.maximum(
                jnp.full((L,), total, jnp.float32),
                jnp.full((L,), 1.0, jnp.float32))
            for g in range(GRP):
                left = zbuf[r, pl.ds(g * L, L)]
                right = zbuf[r, pl.ds(D + g * L, L)]
                cv = fcorr[r, pl.ds(g * L, L)]
                fbuf[r, pl.ds(g * L, L)] = (left + right + cv) * rec
        pltpu.sync_copy(fbuf, out_hbm.at[c, k, pl.ds(s * 8, 8), :])


@jax.jit
def _gavg_pool(f2, idsp, idso, ids2d):
    mesh = plsc.VectorSubcoreMesh(core_axis_name="c", subcore_axis_name="s")
    return pl.kernel(
        _sc_body,
        out_type=jax.ShapeDtypeStruct((NC, NCOMP, NUM_GRAPHS, D),
                                      jnp.float32),
        mesh=mesh,
        compiler_params=pltpu.CompilerParams(
            use_tc_tiling_on_sc=False, needs_layout_passes=False),
        scratch_types=[
            pltpu.VMEM_SHARED((NCOMP * NUM_GRAPHS, D2), jnp.float32),
            pltpu.VMEM_SHARED((NCOMP * NUM_GRAPHS, D), jnp.float32),
            pltpu.VMEM_SHARED((NUM_GRAPHS, L), jnp.float32),
            pltpu.VMEM((NCOMP, SUB_HI, TP), jnp.int32),
            pltpu.VMEM((SUB_HI, TP), jnp.int32),
            pltpu.VMEM((CNT_PER_S, T), jnp.int32),
            pltpu.VMEM((TP, D2), jnp.float32),
            pltpu.VMEM((TP, D2), jnp.float32),
            pltpu.VMEM((NCOMP * NUM_GRAPHS, D), jnp.float32),
            pltpu.VMEM((8, D2), jnp.float32),
            pltpu.VMEM((8, L), jnp.float32),
            pltpu.VMEM((NUM_GRAPHS, L), jnp.float32),
            pltpu.VMEM((1, NUM_GRAPHS), jnp.int32),
            pltpu.VMEM((NCOMP, NUM_GRAPHS), jnp.int32),
            pltpu.VMEM((8, D), jnp.float32),
            pltpu.VMEM((8, D), jnp.float32),
            pltpu.VMEM((8, L), jnp.float32),
            pltpu.SemaphoreType.DMA,
            pltpu.SemaphoreType.DMA,
        ],
    )(f2, idsp, idso, ids2d)


def kernel(features_1, segment_ids):
    f2 = jnp.transpose(features_1, (2, 0, 1)).reshape(NCOMP * NP, D2)
    ids = segment_ids.astype(jnp.int32)
    pad = IDS_P_ROWS * TP - NP
    idsp = jnp.pad(ids[0::2], (0, pad),
                   constant_values=NUM_GRAPHS - 1).reshape(IDS_P_ROWS, TP)
    idso = jnp.pad(ids[1::2], (0, pad),
                   constant_values=NUM_GRAPHS - 1).reshape(IDS_P_ROWS, TP)
    raw = _gavg_pool(f2, idsp, idso, ids.reshape(NSUB, T))
    comb = raw[0] + raw[1]
    return jnp.transpose(comb, (1, 0, 2)).reshape(NUM_GRAPHS, NCOMP * D)
